# baseline (speedup 1.0000x reference)
import functools

import jax
import jax.numpy as jnp
import numpy as np
from jax import lax
from jax.experimental import pallas as pl
from jax.experimental.pallas import tpu as pltpu
from jax.experimental.pallas import tpu_sc as plsc

N_REF = 16384
B = 1024
G = 8
DT = 2.0
STDEV = 1.25

_SIDE = 128

TQ = 512
TK = 4096
NQTOT = B * G

_NC = 2
_NS = 16
_NW = _NC * _NS
_BPW = NQTOT // _NW


def _gl_weights():
    _, w = np.polynomial.legendre.leggauss(G)
    return jnp.asarray(w * (DT / 2.0), dtype=jnp.float32)


def _tables_body(rp_ref, rx_ref, ry_ref, tang_ref, norm_ref, arc_ref):
    rp = rp_ref[...]
    nxt = jnp.concatenate([rp[1:], rp[:1]], axis=0)
    prv = jnp.concatenate([rp[-1:], rp[:-1]], axis=0)
    t = nxt - prv
    tn = jnp.sqrt(jnp.sum(t * t, axis=1, keepdims=True))
    tg = t / tn
    tang_ref[...] = tg
    norm_ref[...] = jnp.concatenate([-tg[:, 1:2], tg[:, 0:1]], axis=1)

    rx = rx_ref[...]
    ry = ry_ref[...]
    ncol_x = jnp.concatenate([rx[1:, 0:1], rx[0:1, 0:1]], axis=0)
    ncol_y = jnp.concatenate([ry[1:, 0:1], ry[0:1, 0:1]], axis=0)
    nxt_x = jnp.concatenate([rx[:, 1:], ncol_x], axis=1)
    nxt_y = jnp.concatenate([ry[:, 1:], ncol_y], axis=1)
    dx = nxt_x - rx
    dy = nxt_y - ry
    seg = jnp.sqrt(dx * dx + dy * dy)
    i0 = lax.broadcasted_iota(jnp.int32, (_SIDE, _SIDE), 0)
    i1 = lax.broadcasted_iota(jnp.int32, (_SIDE, _SIDE), 1)
    upper = (i0 < i1).astype(jnp.float32)
    within = jnp.dot(seg, upper, preferred_element_type=jnp.float32)
    rowsum = jnp.sum(seg, axis=1, keepdims=True)
    lower = (i1 < i0).astype(jnp.float32)
    offs = jnp.dot(lower, rowsum, preferred_element_type=jnp.float32)
    arc_ref[...] = within + offs


def _tables(rp, rx, ry):
    return pl.pallas_call(
        _tables_body,
        out_shape=(
            jax.ShapeDtypeStruct((N_REF, 2), jnp.float32),
            jax.ShapeDtypeStruct((N_REF, 2), jnp.float32),
            jax.ShapeDtypeStruct((_SIDE, _SIDE), jnp.float32),
        ),
    )(rp, rx, ry)


def _argmin_body(qf_ref, qb_ref, rtf_ref, rtb_ref, idx_ref,
                 best_val, best_idx):
    ik = pl.program_id(1)
    nk = pl.num_programs(1)

    qf = qf_ref[...]
    qa = qf[:, 0:1] * qf[:, 0:1] + qf[:, 1:2] * qf[:, 1:2]
    r0 = rtf_ref[0:1, :]
    r1 = rtf_ref[1:2, :]
    pb = r0 * r0 + r1 * r1

    mm2 = jnp.dot(qb_ref[...], rtb_ref[...],
                  preferred_element_type=jnp.float32)
    sq = (qa + pb) - mm2

    m = jnp.min(sq, axis=1, keepdims=True)
    lane = lax.broadcasted_iota(jnp.int32, (TQ, TK), 1).astype(jnp.float32)
    cand = jnp.where(sq == m, lane, jnp.float32(3.0e38))
    li = jnp.min(cand, axis=1, keepdims=True) + jnp.float32(ik * TK)

    @pl.when(ik == 0)
    def _():
        best_val[...] = m
        best_idx[...] = li

    @pl.when(ik > 0)
    def _():
        better = m < best_val[...]
        best_val[...] = jnp.where(better, m, best_val[...])
        best_idx[...] = jnp.where(better, li, best_idx[...])

    @pl.when(ik == nk - 1)
    def _():
        idx_ref[...] = best_idx[...].astype(jnp.int32)


def _argmin(qf, qb, rtf, rtb):
    nq = NQTOT // TQ
    nk = N_REF // TK
    return pl.pallas_call(
        _argmin_body,
        grid=(nq, nk),
        in_specs=[
            pl.BlockSpec((TQ, 2), lambda iq, ik: (iq, 0)),
            pl.BlockSpec((TQ, 2), lambda iq, ik: (iq, 0)),
            pl.BlockSpec((2, TK), lambda iq, ik: (0, ik)),
            pl.BlockSpec((2, TK), lambda iq, ik: (0, ik)),
        ],
        out_specs=pl.BlockSpec((TQ, 1), lambda iq, ik: (iq, 0)),
        out_shape=jax.ShapeDtypeStruct((NQTOT, 1), jnp.int32),
        scratch_shapes=[
            pltpu.VMEM((TQ, 1), jnp.float32),
            pltpu.VMEM((TQ, 1), jnp.float32),
        ],
        compiler_params=pltpu.CompilerParams(
            dimension_semantics=("parallel", "arbitrary"),
        ),
    )(qf, qb, rtf, rtb)


def _gather6_body(rp_hbm, tang_hbm, norm_hbm, arc_hbm, lw_hbm, rw_hbm,
                  idx_hbm,
                  vals_out, tgo_out, nmo_out, arc_out, lw_out, rw_out,
                  idx_v, bv, bt, bn, ba, bl, br, sem):
    wid = lax.axis_index("s") * _NC + lax.axis_index("c")
    base = wid * _BPW
    pltpu.sync_copy(idx_hbm.at[pl.ds(base, _BPW)], idx_v)
    c1 = pltpu.async_copy(rp_hbm.at[idx_v], bv, sem)
    c2 = pltpu.async_copy(tang_hbm.at[idx_v], bt, sem)
    c3 = pltpu.async_copy(norm_hbm.at[idx_v], bn, sem)
    c4 = pltpu.async_copy(arc_hbm.at[idx_v], ba, sem)
    c5 = pltpu.async_copy(lw_hbm.at[idx_v], bl, sem)
    c6 = pltpu.async_copy(rw_hbm.at[idx_v], br, sem)
    c1.wait(); c2.wait(); c3.wait(); c4.wait(); c5.wait(); c6.wait()
    pltpu.sync_copy(bv, vals_out.at[pl.ds(base, _BPW)])
    pltpu.sync_copy(bt, tgo_out.at[pl.ds(base, _BPW)])
    pltpu.sync_copy(bn, nmo_out.at[pl.ds(base, _BPW)])
    pltpu.sync_copy(ba, arc_out.at[pl.ds(base, _BPW)])
    pltpu.sync_copy(bl, lw_out.at[pl.ds(base, _BPW)])
    pltpu.sync_copy(br, rw_out.at[pl.ds(base, _BPW)])


@functools.cache
def _gather_kernel():
    s2 = jax.ShapeDtypeStruct((NQTOT, 2), jnp.float32)
    s1 = jax.ShapeDtypeStruct((NQTOT,), jnp.float32)
    return functools.partial(
        pl.kernel,
        mesh=plsc.VectorSubcoreMesh(core_axis_name="c", subcore_axis_name="s",
                                    num_cores=_NC, num_subcores=_NS),
        out_type=(s2, s2, s2, s1, s1, s1),
        scratch_types=[
            pltpu.VMEM((_BPW,), jnp.int32),
            pltpu.VMEM((_BPW, 2), jnp.float32),
            pltpu.VMEM((_BPW, 2), jnp.float32),
            pltpu.VMEM((_BPW, 2), jnp.float32),
            pltpu.VMEM((_BPW,), jnp.float32),
            pltpu.VMEM((_BPW,), jnp.float32),
            pltpu.VMEM((_BPW,), jnp.float32),
            pltpu.SemaphoreType.DMA,
        ],
        compiler_params=pltpu.CompilerParams(use_tc_tiling_on_sc=False),
    )(_gather6_body)


def _tail_body(q_ref, v_ref, n_ref, lw_ref, rw_ref,
               d_ref, sd_ref, sl_ref, sr_ref):
    sf = jnp.float32(1.0) / (jnp.sqrt(jnp.float32(2.0)) * jnp.float32(STDEV))
    d = q_ref[...] - v_ref[...]
    d_ref[...] = d
    sd = jnp.sum(d * n_ref[...], axis=1, keepdims=True)
    sd_ref[...] = sd
    sl_ref[...] = lax.erf(jnp.maximum(sd - lw_ref[...], 0.0) * sf)
    sr_ref[...] = lax.erf(jnp.maximum(rw_ref[...] - sd, 0.0) * sf)


def _tail(q2, vals2, norm2, lw1, rw1):
    s2 = jax.ShapeDtypeStruct((NQTOT, 2), jnp.float32)
    s1 = jax.ShapeDtypeStruct((NQTOT, 1), jnp.float32)
    return pl.pallas_call(
        _tail_body,
        out_shape=(s2, s1, s1, s1),
    )(q2, vals2, norm2, lw1, rw1)


def _nlnr_body(sl_ref, sr_ref, w_ref, nl_ref, nr_ref):
    w = w_ref[...]
    nl_ref[...] = jnp.exp(-jnp.sum(sl_ref[...] * w, axis=1, keepdims=True))
    nr_ref[...] = jnp.exp(-jnp.sum(sr_ref[...] * w, axis=1, keepdims=True))


def _nlnr(sl8, sr8, w):
    s1 = jax.ShapeDtypeStruct((B, 1), jnp.float32)
    return pl.pallas_call(
        _nlnr_body,
        out_shape=(s1, s1),
    )(sl8, sr8, w)


def kernel(positions, refline_points, left_widths, right_widths):
    q2 = positions.reshape(-1, 2)
    qb = (2.0 * q2).astype(jnp.bfloat16)
    rtf = refline_points.T
    rtb = rtf.astype(jnp.bfloat16)
    rx = refline_points[:, 0].reshape(_SIDE, _SIDE)
    ry = refline_points[:, 1].reshape(_SIDE, _SIDE)

    tang_t, norm_t, arc2d = _tables(refline_points, rx, ry)
    arc_t = arc2d.reshape(N_REF)

    idx2d = _argmin(q2, qb, rtf, rtb)

    vals2, tang2, norm2, arcg, lwg, rwg = _gather_kernel()(
        refline_points, tang_t, norm_t, arc_t, left_widths, right_widths,
        idx2d.reshape(NQTOT))

    dxy, sd1, sl1, sr1 = _tail(q2, vals2, norm2,
                               lwg.reshape(NQTOT, 1), rwg.reshape(NQTOT, 1))

    sl8 = sl1.reshape(B, G)
    sr8 = sr1.reshape(B, G)
    nl, nr = _nlnr(sl8, sr8, _gl_weights().reshape(1, G))

    return (arcg.reshape(B, G), vals2.reshape(B, G, 2),
            tang2.reshape(B, G, 2), norm2.reshape(B, G, 2),
            dxy.reshape(B, G, 2), sd1.reshape(B, G),
            lwg.reshape(B, G), rwg.reshape(B, G), sl8, sr8,
            nl.reshape(B), nr.reshape(B))


# speedup vs baseline: 1.0688x; 1.0688x over previous
import functools

import jax
import jax.numpy as jnp
import numpy as np
from jax import lax
from jax.experimental import pallas as pl
from jax.experimental.pallas import tpu as pltpu
from jax.experimental.pallas import tpu_sc as plsc

N_REF = 16384
B = 1024
G = 8
DT = 2.0
STDEV = 1.25

_SIDE = 128

TQ = 512
TK = 4096
NQTOT = B * G

_NC = 2
_NS = 16
_NW = _NC * _NS
_BPW = NQTOT // _NW
_TD = 16


def _gl_weights():
    _, w = np.polynomial.legendre.leggauss(G)
    return jnp.asarray(w * (DT / 2.0), dtype=jnp.float32)


def _tables_body(rp_ref, rx_ref, ry_ref, lw_ref, rw_ref, tab_ref, arc_ref):
    rp = rp_ref[...]
    nxt = jnp.concatenate([rp[1:], rp[:1]], axis=0)
    prv = jnp.concatenate([rp[-1:], rp[:-1]], axis=0)
    t = nxt - prv
    tn = jnp.sqrt(jnp.sum(t * t, axis=1, keepdims=True))
    tg = t / tn
    nm = jnp.concatenate([-tg[:, 1:2], tg[:, 0:1]], axis=1)
    tab_ref[:, 0:2] = rp
    tab_ref[:, 2:4] = tg
    tab_ref[:, 4:6] = nm
    tab_ref[:, 6:7] = lw_ref[...]
    tab_ref[:, 7:8] = rw_ref[...]
    tab_ref[:, 8:16] = jnp.zeros((N_REF, 8), jnp.float32)

    rx = rx_ref[...]
    ry = ry_ref[...]
    ncol_x = jnp.concatenate([rx[1:, 0:1], rx[0:1, 0:1]], axis=0)
    ncol_y = jnp.concatenate([ry[1:, 0:1], ry[0:1, 0:1]], axis=0)
    nxt_x = jnp.concatenate([rx[:, 1:], ncol_x], axis=1)
    nxt_y = jnp.concatenate([ry[:, 1:], ncol_y], axis=1)
    dx = nxt_x - rx
    dy = nxt_y - ry
    seg = jnp.sqrt(dx * dx + dy * dy)
    i0 = lax.broadcasted_iota(jnp.int32, (_SIDE, _SIDE), 0)
    i1 = lax.broadcasted_iota(jnp.int32, (_SIDE, _SIDE), 1)
    upper = (i0 < i1).astype(jnp.float32)
    within = jnp.dot(seg, upper, preferred_element_type=jnp.float32)
    rowsum = jnp.sum(seg, axis=1, keepdims=True)
    lower = (i1 < i0).astype(jnp.float32)
    offs = jnp.dot(lower, rowsum, preferred_element_type=jnp.float32)
    arc_ref[...] = within + offs


def _tables(rp, rx, ry, lw1, rw1):
    return pl.pallas_call(
        _tables_body,
        out_shape=(
            jax.ShapeDtypeStruct((N_REF, _TD), jnp.float32),
            jax.ShapeDtypeStruct((_SIDE, _SIDE), jnp.float32),
        ),
    )(rp, rx, ry, lw1, rw1)


def _argmin_body(qf_ref, qb_ref, rtf_ref, rtb_ref, idx_ref,
                 best_val, best_idx):
    ik = pl.program_id(1)
    nk = pl.num_programs(1)

    qf = qf_ref[...]
    qa = qf[:, 0:1] * qf[:, 0:1] + qf[:, 1:2] * qf[:, 1:2]
    r0 = rtf_ref[0:1, :]
    r1 = rtf_ref[1:2, :]
    pb = r0 * r0 + r1 * r1

    mm2 = jnp.dot(qb_ref[...], rtb_ref[...],
                  preferred_element_type=jnp.float32)
    sq = (qa + pb) - mm2

    m = jnp.min(sq, axis=1, keepdims=True)
    lane = lax.broadcasted_iota(jnp.int32, (TQ, TK), 1).astype(jnp.float32)
    cand = jnp.where(sq == m, lane, jnp.float32(3.0e38))
    li = jnp.min(cand, axis=1, keepdims=True) + jnp.float32(ik * TK)

    @pl.when(ik == 0)
    def _():
        best_val[...] = m
        best_idx[...] = li

    @pl.when(ik > 0)
    def _():
        better = m < best_val[...]
        best_val[...] = jnp.where(better, m, best_val[...])
        best_idx[...] = jnp.where(better, li, best_idx[...])

    @pl.when(ik == nk - 1)
    def _():
        idx_ref[...] = best_idx[...].astype(jnp.int32)


def _argmin(qf, qb, rtf, rtb):
    nq = NQTOT // TQ
    nk = N_REF // TK
    return pl.pallas_call(
        _argmin_body,
        grid=(nq, nk),
        in_specs=[
            pl.BlockSpec((TQ, 2), lambda iq, ik: (iq, 0)),
            pl.BlockSpec((TQ, 2), lambda iq, ik: (iq, 0)),
            pl.BlockSpec((2, TK), lambda iq, ik: (0, ik)),
            pl.BlockSpec((2, TK), lambda iq, ik: (0, ik)),
        ],
        out_specs=pl.BlockSpec((TQ, 1), lambda iq, ik: (iq, 0)),
        out_shape=jax.ShapeDtypeStruct((NQTOT, 1), jnp.int32),
        scratch_shapes=[
            pltpu.VMEM((TQ, 1), jnp.float32),
            pltpu.VMEM((TQ, 1), jnp.float32),
        ],
        compiler_params=pltpu.CompilerParams(
            dimension_semantics=("parallel", "arbitrary"),
        ),
    )(qf, qb, rtf, rtb)


def _gather_body(tab_hbm, arc_hbm, idx_hbm, rows_out, arc_out,
                 idx_v, rows_v, arc_v, sem):
    wid = lax.axis_index("s") * _NC + lax.axis_index("c")
    base = wid * _BPW
    pltpu.sync_copy(idx_hbm.at[pl.ds(base, _BPW)], idx_v)
    c1 = pltpu.async_copy(tab_hbm.at[idx_v], rows_v, sem)
    c2 = pltpu.async_copy(arc_hbm.at[idx_v], arc_v, sem)
    c1.wait()
    c2.wait()
    pltpu.sync_copy(rows_v, rows_out.at[pl.ds(base, _BPW)])
    pltpu.sync_copy(arc_v, arc_out.at[pl.ds(base, _BPW)])


@functools.cache
def _gather_kernel():
    return functools.partial(
        pl.kernel,
        mesh=plsc.VectorSubcoreMesh(core_axis_name="c", subcore_axis_name="s",
                                    num_cores=_NC, num_subcores=_NS),
        out_type=(jax.ShapeDtypeStruct((NQTOT, _TD), jnp.float32),
                  jax.ShapeDtypeStruct((NQTOT,), jnp.float32)),
        scratch_types=[
            pltpu.VMEM((_BPW,), jnp.int32),
            pltpu.VMEM((_BPW, _TD), jnp.float32),
            pltpu.VMEM((_BPW,), jnp.float32),
            pltpu.SemaphoreType.DMA,
        ],
        compiler_params=pltpu.CompilerParams(use_tc_tiling_on_sc=False),
    )(_gather_body)


def _tail_body(q_ref, g_ref, d_ref, sd_ref, sl_ref, sr_ref):
    sf = jnp.float32(1.0) / (jnp.sqrt(jnp.float32(2.0)) * jnp.float32(STDEV))
    g = g_ref[...]
    d = q_ref[...] - g[:, 0:2]
    d_ref[...] = d
    sd = jnp.sum(d * g[:, 4:6], axis=1, keepdims=True)
    sd_ref[...] = sd
    sl_ref[...] = lax.erf(jnp.maximum(sd - g[:, 6:7], 0.0) * sf)
    sr_ref[...] = lax.erf(jnp.maximum(g[:, 7:8] - sd, 0.0) * sf)


def _tail(q2, g):
    s2 = jax.ShapeDtypeStruct((NQTOT, 2), jnp.float32)
    s1 = jax.ShapeDtypeStruct((NQTOT, 1), jnp.float32)
    return pl.pallas_call(
        _tail_body,
        out_shape=(s2, s1, s1, s1),
    )(q2, g)


def _nlnr_body(sl_ref, sr_ref, w_ref, nl_ref, nr_ref):
    w = w_ref[...]
    nl_ref[...] = jnp.exp(-jnp.sum(sl_ref[...] * w, axis=1, keepdims=True))
    nr_ref[...] = jnp.exp(-jnp.sum(sr_ref[...] * w, axis=1, keepdims=True))


def _nlnr(sl8, sr8, w):
    s1 = jax.ShapeDtypeStruct((B, 1), jnp.float32)
    return pl.pallas_call(
        _nlnr_body,
        out_shape=(s1, s1),
    )(sl8, sr8, w)


def kernel(positions, refline_points, left_widths, right_widths):
    q2 = positions.reshape(-1, 2)
    qb = (2.0 * q2).astype(jnp.bfloat16)
    rtf = refline_points.T
    rtb = rtf.astype(jnp.bfloat16)
    rx = refline_points[:, 0].reshape(_SIDE, _SIDE)
    ry = refline_points[:, 1].reshape(_SIDE, _SIDE)

    table, arc2d = _tables(refline_points, rx, ry,
                           left_widths.reshape(N_REF, 1),
                           right_widths.reshape(N_REF, 1))
    arc_t = arc2d.reshape(N_REF)

    idx2d = _argmin(q2, qb, rtf, rtb)

    g, arcg = _gather_kernel()(table, arc_t, idx2d.reshape(NQTOT))

    dxy, sd1, sl1, sr1 = _tail(q2, g)

    sl8 = sl1.reshape(B, G)
    sr8 = sr1.reshape(B, G)
    nl, nr = _nlnr(sl8, sr8, _gl_weights().reshape(1, G))

    return (arcg.reshape(B, G), g[:, 0:2].reshape(B, G, 2),
            g[:, 2:4].reshape(B, G, 2), g[:, 4:6].reshape(B, G, 2),
            dxy.reshape(B, G, 2), sd1.reshape(B, G),
            g[:, 6].reshape(B, G), g[:, 7].reshape(B, G), sl8, sr8,
            nl.reshape(B), nr.reshape(B))
